# Initial kernel scaffold; baseline (speedup 1.0000x reference)
import jax
import jax.numpy as jnp
from jax.experimental import pallas as pl


def kernel(features, user_id_preference, edge_index, edge_weight, W, b):
    raise NotImplementedError("write your pallas kernel here")


# speedup vs baseline: 2.5415x; 2.5415x over previous
import functools

import jax
import jax.numpy as jnp
from jax import lax
from jax.experimental import pallas as pl
from jax.experimental.pallas import tpu as pltpu
from jax.experimental.pallas import tpu_sc as plsc

NUM_USER = 10000
NUM_ITEM = 40000
N_NODES = NUM_USER + NUM_ITEM
DIM_FEAT = 512
DIM_LATENT = 64
N_EDGES = 800000
N_LAYERS = 3

_NC = 2
_NS = 16
_N_OWN = N_NODES // _NC
_TRASH = 64
_SLAB = 1568
_ACC_ROWS = _NS * _SLAB
_W = 128
_N_WIN = N_EDGES // _W


def _propagate(emb, src, dst, wgt):
    mesh = plsc.VectorSubcoreMesh(
        core_axis_name="c", subcore_axis_name="s",
        num_cores=_NC, num_subcores=_NS)

    @functools.partial(
        pl.kernel,
        out_type=jax.ShapeDtypeStruct((N_NODES, DIM_LATENT), jnp.float32),
        mesh=mesh,
        compiler_params=pltpu.CompilerParams(use_tc_tiling_on_sc=False),
        scratch_types=[
            pltpu.VMEM_SHARED((_ACC_ROWS, DIM_LATENT), jnp.float32),
            pltpu.VMEM((1, _W), jnp.int32),
            pltpu.VMEM((1, _W), jnp.int32),
            pltpu.VMEM((1, _W), jnp.float32),
            pltpu.VMEM((_W, DIM_LATENT), jnp.float32),
            pltpu.VMEM((_W, DIM_LATENT), jnp.float32),
            pltpu.SemaphoreType.DMA,
        ],
    )
    def k(emb_h, src_h, dst_h, wgt_h, out_h, acc, srcb, dstb, wgtb, rows,
          zbuf, sem):
        c = lax.axis_index("c")
        s = lax.axis_index("s")
        lo = c * _N_OWN
        base = s * _SLAB

        def _zrow(r, carry):
            for q in range(DIM_LATENT // 16):
                zbuf[r, pl.ds(q * 16, 16)] = jnp.zeros((16,), jnp.float32)
            return carry
        lax.fori_loop(0, _W, _zrow, 0)

        def _zcp(i, carry):
            pltpu.sync_copy(zbuf, acc.at[pl.ds(base + i * _W, _W)])
            return carry
        lax.fori_loop(0, _SLAB // _W, _zcp, 0)
        pltpu.sync_copy(zbuf.at[pl.ds(0, _SLAB % _W)],
                        acc.at[pl.ds(base + (_SLAB // _W) * _W, _SLAB % _W)])
        plsc.subcore_barrier()

        n_my = (_N_WIN - s + _NS - 1) // _NS

        def _win(kk, carry):
            e0 = (s + kk * _NS) * _W
            pltpu.sync_copy(src_h.at[pl.ds(e0, _W)], srcb.at[0])
            pltpu.sync_copy(dst_h.at[pl.ds(e0, _W)], dstb.at[0])
            pltpu.sync_copy(wgt_h.at[pl.ds(e0, _W)], wgtb.at[0])
            pltpu.async_copy(emb_h.at[srcb.at[0]], rows, sem).wait()
            for t in range(_W // 16):
                d = dstb[0, pl.ds(t * 16, 16)]
                own = (d >= lo) & (d < lo + _N_OWN)
                loc = jnp.where(own, d - lo, _N_OWN + (d & (_TRASH - 1)))
                dstb[0, pl.ds(t * 16, 16)] = loc
            def _wmul(g, carry2):
                w16 = wgtb[0, pl.ds(g * 16, 16)]
                for eo in range(16):
                    e = g * 16 + eo
                    wv = w16[eo]
                    for q in range(DIM_LATENT // 16):
                        rows[e, pl.ds(q * 16, 16)] = (
                            rows[e, pl.ds(q * 16, 16)] * wv)
                return carry2
            lax.fori_loop(0, _W // 16, _wmul, 0)
            pltpu.sync_copy(rows, acc.at[dstb.at[0]], add=True)
            return carry
        lax.fori_loop(0, n_my, _win, 0)
        plsc.subcore_barrier()

        n_full = jnp.where(s < _NS - 1, 12, 11)

        def _wb(i, carry):
            pltpu.sync_copy(acc.at[pl.ds(base + i * _W, _W)],
                            out_h.at[pl.ds(lo + base + i * _W, _W)])
            return carry
        lax.fori_loop(0, n_full, _wb, 0)

        @pl.when(s < _NS - 1)
        def _tail_a():
            pltpu.sync_copy(acc.at[pl.ds(base + 12 * _W, 32)],
                            out_h.at[pl.ds(lo + base + 12 * _W, 32)])

        @pl.when(s == _NS - 1)
        def _tail_b():
            pltpu.sync_copy(acc.at[pl.ds(base + 11 * _W, 72)],
                            out_h.at[pl.ds(lo + base + 11 * _W, 72)])

    return k(emb, src, dst, wgt)


_BM = 2000


def _item_body(f_ref, wt_ref, b_ref, o_ref):
    x = jnp.dot(f_ref[...], wt_ref[...], preferred_element_type=jnp.float32)
    x = x + b_ref[...]
    norm = jnp.sqrt(jnp.sum(x * x, axis=1, keepdims=True))
    o_ref[...] = x / jnp.maximum(norm, 1e-12)


def _user_body(u_ref, o_ref):
    x = u_ref[...]
    norm = jnp.sqrt(jnp.sum(x * x, axis=1, keepdims=True))
    o_ref[...] = x / jnp.maximum(norm, 1e-12)


def _mean_body(a_ref, b_ref, c_ref, d_ref, o_ref):
    o_ref[...] = 0.25 * (a_ref[...] + b_ref[...] + c_ref[...] + d_ref[...])


def kernel(features, user_id_preference, edge_index, edge_weight, W, b):
    Wt = W.T
    b2 = b.reshape(1, DIM_LATENT)

    items0 = pl.pallas_call(
        _item_body,
        grid=(NUM_ITEM // _BM,),
        in_specs=[
            pl.BlockSpec((_BM, DIM_FEAT), lambda i: (i, 0)),
            pl.BlockSpec((DIM_FEAT, DIM_LATENT), lambda i: (0, 0)),
            pl.BlockSpec((1, DIM_LATENT), lambda i: (0, 0)),
        ],
        out_specs=pl.BlockSpec((_BM, DIM_LATENT), lambda i: (i, 0)),
        out_shape=jax.ShapeDtypeStruct((NUM_ITEM, DIM_LATENT), jnp.float32),
    )(features, Wt, b2)

    users0 = pl.pallas_call(
        _user_body,
        out_shape=jax.ShapeDtypeStruct((NUM_USER, DIM_LATENT), jnp.float32),
    )(user_id_preference)

    emb0 = jnp.concatenate([users0, items0], axis=0)

    src = edge_index[0]
    dst = edge_index[1]
    emb1 = _propagate(emb0, src, dst, edge_weight)
    emb2 = _propagate(emb1, src, dst, edge_weight)
    emb3 = _propagate(emb2, src, dst, edge_weight)

    _BR = 2000
    light = pl.pallas_call(
        _mean_body,
        grid=(N_NODES // _BR,),
        in_specs=[pl.BlockSpec((_BR, DIM_LATENT), lambda i: (i, 0))] * 4,
        out_specs=pl.BlockSpec((_BR, DIM_LATENT), lambda i: (i, 0)),
        out_shape=jax.ShapeDtypeStruct((N_NODES, DIM_LATENT), jnp.float32),
    )(emb0, emb1, emb2, emb3)

    return (light[:NUM_USER], light[NUM_USER:])


# speedup vs baseline: 4.5929x; 1.8071x over previous
import functools

import jax
import jax.numpy as jnp
from jax import lax
from jax.experimental import pallas as pl
from jax.experimental.pallas import tpu as pltpu
from jax.experimental.pallas import tpu_sc as plsc

NUM_USER = 10000
NUM_ITEM = 40000
N_NODES = NUM_USER + NUM_ITEM
DIM_FEAT = 512
DIM_LATENT = 64
N_EDGES = 800000
N_LAYERS = 3

_NC = 2
_NS = 16
_N_OWN = N_NODES // _NC
_TRASH = 64
_SLAB = 1568
_ACC_ROWS = _NS * _SLAB
_W = 128
_N_WIN = N_EDGES // _W
_N_SUP = N_EDGES // (2 * _W)


def _propagate(emb, src3, dst3, wgt3):
    mesh = plsc.VectorSubcoreMesh(
        core_axis_name="c", subcore_axis_name="s",
        num_cores=_NC, num_subcores=_NS)

    @functools.partial(
        pl.kernel,
        out_type=jax.ShapeDtypeStruct((N_NODES, DIM_LATENT), jnp.float32),
        mesh=mesh,
        compiler_params=pltpu.CompilerParams(use_tc_tiling_on_sc=False),
        scratch_types=[
            pltpu.VMEM_SHARED((_ACC_ROWS, DIM_LATENT), jnp.float32),
            pltpu.VMEM((2, 2, _W), jnp.int32),
            pltpu.VMEM((2, 2, _W), jnp.int32),
            pltpu.VMEM((2, 2, _W), jnp.float32),
            pltpu.VMEM((2, _W, DIM_LATENT), jnp.float32),
            pltpu.VMEM((_W, DIM_LATENT), jnp.float32),
            pltpu.SemaphoreType.DMA,
            pltpu.SemaphoreType.DMA,
            pltpu.SemaphoreType.DMA,
            pltpu.SemaphoreType.DMA,
        ],
    )
    def k(emb_h, src_h, dst_h, wgt_h, out_h, acc, srcb, dstb, wgtb, rows,
          zbuf, semg0, semg1, sems0, sems1):
        semg = (semg0, semg1)
        c = lax.axis_index("c")
        s = lax.axis_index("s")
        lo = c * _N_OWN
        base = s * _SLAB

        def _zrow(r, carry):
            for q in range(DIM_LATENT // 16):
                zbuf[r, pl.ds(q * 16, 16)] = jnp.zeros((16,), jnp.float32)
            return carry
        lax.fori_loop(0, _W, _zrow, 0)

        def _zcp(i, carry):
            pltpu.sync_copy(zbuf, acc.at[pl.ds(base + i * _W, _W)])
            return carry
        lax.fori_loop(0, _SLAB // _W, _zcp, 0)
        pltpu.sync_copy(zbuf.at[pl.ds(0, _SLAB % _W)],
                        acc.at[pl.ds(base + (_SLAB // _W) * _W, _SLAB % _W)])
        plsc.subcore_barrier()

        n_sup = (_N_SUP - s + _NS - 1) // _NS

        def _idx_load(kk2, b2):
            sup = s + kk2 * _NS
            pltpu.sync_copy(src_h.at[sup], srcb.at[b2])
            pltpu.sync_copy(dst_h.at[sup], dstb.at[b2])
            pltpu.sync_copy(wgt_h.at[sup], wgtb.at[b2])

        def _g_issue(b2, j):
            pltpu.async_copy(emb_h.at[srcb.at[b2, j]], rows.at[j], semg[j])

        def _g_wait(b2, j):
            pltpu.make_async_copy(
                emb_h.at[srcb.at[b2, j]], rows.at[j], semg[j]).wait()

        def _compute(b2, j):
            for t in range(_W // 16):
                d = dstb[b2, j, pl.ds(t * 16, 16)]
                own = (d >= lo) & (d < lo + _N_OWN)
                loc = jnp.where(own, d - lo, _N_OWN + (d & (_TRASH - 1)))
                dstb[b2, j, pl.ds(t * 16, 16)] = loc

            def _wmul(g, carry2):
                w16 = wgtb[b2, j, pl.ds(g * 16, 16)]
                for eo in range(16):
                    e = g * 16 + eo
                    wv = w16[eo]
                    for q in range(DIM_LATENT // 16):
                        rows[j, e, pl.ds(q * 16, 16)] = (
                            rows[j, e, pl.ds(q * 16, 16)] * wv)
                return carry2
            lax.fori_loop(0, _W // 16, _wmul, 0)

        _idx_load(0, 0)
        _g_issue(0, 0)

        def _sup2(kk4, carry):
            for b2 in range(2):
                kk2 = kk4 * 2 + b2

                @pl.when(kk2 < n_sup)
                def _():
                    @pl.when(kk2 + 1 < n_sup)
                    def _():
                        _idx_load(kk2 + 1, 1 - b2)
                    _g_wait(b2, 0)
                    _g_issue(b2, 1)
                    _compute(b2, 0)
                    sd0 = pltpu.async_copy(
                        rows.at[0], acc.at[dstb.at[b2, 0]], sems0, add=True)
                    _g_wait(b2, 1)
                    _compute(b2, 1)
                    sd0.wait()

                    @pl.when(kk2 + 1 < n_sup)
                    def _():
                        _g_issue(1 - b2, 0)
                    sd1 = pltpu.async_copy(
                        rows.at[1], acc.at[dstb.at[b2, 1]], sems1, add=True)
                    sd1.wait()
            return carry
        lax.fori_loop(0, (n_sup + 1) // 2, _sup2, 0)
        plsc.subcore_barrier()

        n_full = jnp.where(s < _NS - 1, 12, 11)

        def _wb(i, carry):
            pltpu.sync_copy(acc.at[pl.ds(base + i * _W, _W)],
                            out_h.at[pl.ds(lo + base + i * _W, _W)])
            return carry
        lax.fori_loop(0, n_full, _wb, 0)

        @pl.when(s < _NS - 1)
        def _tail_a():
            pltpu.sync_copy(acc.at[pl.ds(base + 12 * _W, 32)],
                            out_h.at[pl.ds(lo + base + 12 * _W, 32)])

        @pl.when(s == _NS - 1)
        def _tail_b():
            pltpu.sync_copy(acc.at[pl.ds(base + 11 * _W, 72)],
                            out_h.at[pl.ds(lo + base + 11 * _W, 72)])

    return k(emb, src3, dst3, wgt3)


_BM = 2000


def _item_body(f_ref, wt_ref, b_ref, o_ref):
    x = jnp.dot(f_ref[...], wt_ref[...], preferred_element_type=jnp.float32)
    x = x + b_ref[...]
    norm = jnp.sqrt(jnp.sum(x * x, axis=1, keepdims=True))
    o_ref[...] = x / jnp.maximum(norm, 1e-12)


def _user_body(u_ref, o_ref):
    x = u_ref[...]
    norm = jnp.sqrt(jnp.sum(x * x, axis=1, keepdims=True))
    o_ref[...] = x / jnp.maximum(norm, 1e-12)


def _mean_body(a_ref, b_ref, c_ref, d_ref, o_ref):
    o_ref[...] = 0.25 * (a_ref[...] + b_ref[...] + c_ref[...] + d_ref[...])


def kernel(features, user_id_preference, edge_index, edge_weight, W, b):
    Wt = W.T
    b2 = b.reshape(1, DIM_LATENT)

    items0 = pl.pallas_call(
        _item_body,
        grid=(NUM_ITEM // _BM,),
        in_specs=[
            pl.BlockSpec((_BM, DIM_FEAT), lambda i: (i, 0)),
            pl.BlockSpec((DIM_FEAT, DIM_LATENT), lambda i: (0, 0)),
            pl.BlockSpec((1, DIM_LATENT), lambda i: (0, 0)),
        ],
        out_specs=pl.BlockSpec((_BM, DIM_LATENT), lambda i: (i, 0)),
        out_shape=jax.ShapeDtypeStruct((NUM_ITEM, DIM_LATENT), jnp.float32),
    )(features, Wt, b2)

    users0 = pl.pallas_call(
        _user_body,
        out_shape=jax.ShapeDtypeStruct((NUM_USER, DIM_LATENT), jnp.float32),
    )(user_id_preference)

    emb0 = jnp.concatenate([users0, items0], axis=0)

    src3 = edge_index[0].reshape(_N_SUP, 2, _W)
    dst3 = edge_index[1].reshape(_N_SUP, 2, _W)
    wgt3 = edge_weight.reshape(_N_SUP, 2, _W)
    emb1 = _propagate(emb0, src3, dst3, wgt3)
    emb2 = _propagate(emb1, src3, dst3, wgt3)
    emb3 = _propagate(emb2, src3, dst3, wgt3)

    _BR = 2000
    light = pl.pallas_call(
        _mean_body,
        grid=(N_NODES // _BR,),
        in_specs=[pl.BlockSpec((_BR, DIM_LATENT), lambda i: (i, 0))] * 4,
        out_specs=pl.BlockSpec((_BR, DIM_LATENT), lambda i: (i, 0)),
        out_shape=jax.ShapeDtypeStruct((N_NODES, DIM_LATENT), jnp.float32),
    )(emb0, emb1, emb2, emb3)

    return (light[:NUM_USER], light[NUM_USER:])


# speedup vs baseline: 5.2165x; 1.1358x over previous
import functools

import jax
import jax.numpy as jnp
from jax import lax
from jax.experimental import pallas as pl
from jax.experimental.pallas import tpu as pltpu
from jax.experimental.pallas import tpu_sc as plsc

NUM_USER = 10000
NUM_ITEM = 40000
N_NODES = NUM_USER + NUM_ITEM
DIM_FEAT = 512
DIM_LATENT = 64
N_EDGES = 800000
N_LAYERS = 3

_NC = 2
_NS = 16
_COLS = DIM_LATENT // _NC
_SLAB = 3136
_ACC_ROWS = _NS * _SLAB
_W = 128
_N_SUP = N_EDGES // (2 * _W)


def _propagate(emb, src3, dst3, wgt3):
    mesh = plsc.VectorSubcoreMesh(
        core_axis_name="c", subcore_axis_name="s",
        num_cores=_NC, num_subcores=_NS)

    @functools.partial(
        pl.kernel,
        out_type=jax.ShapeDtypeStruct((_NC, N_NODES, _COLS), jnp.float32),
        mesh=mesh,
        compiler_params=pltpu.CompilerParams(use_tc_tiling_on_sc=False),
        scratch_types=[
            pltpu.VMEM_SHARED((_ACC_ROWS, _COLS), jnp.float32),
            pltpu.VMEM((2, 2, _W), jnp.int32),
            pltpu.VMEM((2, 2, _W), jnp.int32),
            pltpu.VMEM((2, 2, _W), jnp.float32),
            pltpu.VMEM((2, _W, _COLS), jnp.float32),
            pltpu.VMEM((_W, _COLS), jnp.float32),
            pltpu.SemaphoreType.DMA,
            pltpu.SemaphoreType.DMA,
            pltpu.SemaphoreType.DMA,
            pltpu.SemaphoreType.DMA,
        ],
    )
    def k(emb_h, src_h, dst_h, wgt_h, out_h, acc, srcb, dstb, wgtb, rows,
          zbuf, semg0, semg1, sems0, sems1):
        semg = (semg0, semg1)
        c = lax.axis_index("c")
        s = lax.axis_index("s")
        my_emb = emb_h.at[c]
        my_out = out_h.at[c]
        base = s * _SLAB

        def _zrow(r, carry):
            for q in range(_COLS // 16):
                zbuf[r, pl.ds(q * 16, 16)] = jnp.zeros((16,), jnp.float32)
            return carry
        lax.fori_loop(0, _W, _zrow, 0)

        def _zcp(i, carry):
            pltpu.sync_copy(zbuf, acc.at[pl.ds(base + i * _W, _W)])
            return carry
        lax.fori_loop(0, _SLAB // _W, _zcp, 0)
        pltpu.sync_copy(zbuf.at[pl.ds(0, _SLAB % _W)],
                        acc.at[pl.ds(base + (_SLAB // _W) * _W, _SLAB % _W)])
        plsc.subcore_barrier()

        n_sup = (_N_SUP - s + _NS - 1) // _NS

        def _idx_load(kk2, b2):
            sup = s + kk2 * _NS
            pltpu.sync_copy(src_h.at[sup], srcb.at[b2])
            pltpu.sync_copy(dst_h.at[sup], dstb.at[b2])
            pltpu.sync_copy(wgt_h.at[sup], wgtb.at[b2])

        def _g_issue(b2, j):
            pltpu.async_copy(my_emb.at[srcb.at[b2, j]], rows.at[j], semg[j])

        def _g_wait(b2, j):
            pltpu.make_async_copy(
                my_emb.at[srcb.at[b2, j]], rows.at[j], semg[j]).wait()

        def _compute(b2, j):
            def _wmul(g, carry2):
                w16 = wgtb[b2, j, pl.ds(g * 16, 16)]
                for eo in range(16):
                    e = g * 16 + eo
                    wv = w16[eo]
                    for q in range(_COLS // 16):
                        rows[j, e, pl.ds(q * 16, 16)] = (
                            rows[j, e, pl.ds(q * 16, 16)] * wv)
                return carry2
            lax.fori_loop(0, _W // 16, _wmul, 0)

        _idx_load(0, 0)
        _g_issue(0, 0)

        def _sup2(kk4, carry):
            for b2 in range(2):
                kk2 = kk4 * 2 + b2

                @pl.when(kk2 < n_sup)
                def _():
                    @pl.when(kk2 + 1 < n_sup)
                    def _():
                        _idx_load(kk2 + 1, 1 - b2)
                    _g_wait(b2, 0)
                    _g_issue(b2, 1)
                    _compute(b2, 0)
                    sd0 = pltpu.async_copy(
                        rows.at[0], acc.at[dstb.at[b2, 0]], sems0, add=True)
                    _g_wait(b2, 1)
                    _compute(b2, 1)
                    sd0.wait()

                    @pl.when(kk2 + 1 < n_sup)
                    def _():
                        _g_issue(1 - b2, 0)
                    sd1 = pltpu.async_copy(
                        rows.at[1], acc.at[dstb.at[b2, 1]], sems1, add=True)
                    sd1.wait()
            return carry
        lax.fori_loop(0, (n_sup + 1) // 2, _sup2, 0)
        plsc.subcore_barrier()

        n_full = jnp.where(s < _NS - 1, _SLAB // _W, 23)

        def _wb(i, carry):
            pltpu.sync_copy(acc.at[pl.ds(base + i * _W, _W)],
                            my_out.at[pl.ds(base + i * _W, _W)])
            return carry
        lax.fori_loop(0, n_full, _wb, 0)

        @pl.when(s < _NS - 1)
        def _tail_a():
            pltpu.sync_copy(acc.at[pl.ds(base + 24 * _W, 64)],
                            my_out.at[pl.ds(base + 24 * _W, 64)])

        @pl.when(s == _NS - 1)
        def _tail_b():
            pltpu.sync_copy(acc.at[pl.ds(base + 23 * _W, 16)],
                            my_out.at[pl.ds(base + 23 * _W, 16)])

    return k(emb, src3, dst3, wgt3)


_BM = 2000


def _item_body(f_ref, wt_ref, b_ref, o_ref):
    x = jnp.dot(f_ref[...], wt_ref[...], preferred_element_type=jnp.float32)
    x = x + b_ref[...]
    norm = jnp.sqrt(jnp.sum(x * x, axis=1, keepdims=True))
    o_ref[...] = x / jnp.maximum(norm, 1e-12)


def _user_body(u_ref, o_ref):
    x = u_ref[...]
    norm = jnp.sqrt(jnp.sum(x * x, axis=1, keepdims=True))
    o_ref[...] = x / jnp.maximum(norm, 1e-12)


def _mean_body(a_ref, b_ref, c_ref, d_ref, o_ref):
    o_ref[...] = 0.25 * (a_ref[...] + b_ref[...] + c_ref[...] + d_ref[...])


def kernel(features, user_id_preference, edge_index, edge_weight, W, b):
    Wt = W.T
    b2 = b.reshape(1, DIM_LATENT)

    items0 = pl.pallas_call(
        _item_body,
        grid=(NUM_ITEM // _BM,),
        in_specs=[
            pl.BlockSpec((_BM, DIM_FEAT), lambda i: (i, 0)),
            pl.BlockSpec((DIM_FEAT, DIM_LATENT), lambda i: (0, 0)),
            pl.BlockSpec((1, DIM_LATENT), lambda i: (0, 0)),
        ],
        out_specs=pl.BlockSpec((_BM, DIM_LATENT), lambda i: (i, 0)),
        out_shape=jax.ShapeDtypeStruct((NUM_ITEM, DIM_LATENT), jnp.float32),
    )(features, Wt, b2)

    users0 = pl.pallas_call(
        _user_body,
        out_shape=jax.ShapeDtypeStruct((NUM_USER, DIM_LATENT), jnp.float32),
    )(user_id_preference)

    emb0 = jnp.concatenate([users0, items0], axis=0)
    emb0_ab = jnp.stack([emb0[:, :_COLS], emb0[:, _COLS:]], axis=0)

    src3 = edge_index[0].reshape(_N_SUP, 2, _W)
    dst3 = edge_index[1].reshape(_N_SUP, 2, _W)
    wgt3 = edge_weight.reshape(_N_SUP, 2, _W)
    emb1 = _propagate(emb0_ab, src3, dst3, wgt3)
    emb2 = _propagate(emb1, src3, dst3, wgt3)
    emb3 = _propagate(emb2, src3, dst3, wgt3)

    _BR = 2000
    m2 = pl.pallas_call(
        _mean_body,
        grid=(_NC * N_NODES // _BR,),
        in_specs=[pl.BlockSpec((_BR, _COLS), lambda i: (i, 0))] * 4,
        out_specs=pl.BlockSpec((_BR, _COLS), lambda i: (i, 0)),
        out_shape=jax.ShapeDtypeStruct((_NC * N_NODES, _COLS), jnp.float32),
    )(emb0_ab.reshape(_NC * N_NODES, _COLS),
      emb1.reshape(_NC * N_NODES, _COLS),
      emb2.reshape(_NC * N_NODES, _COLS),
      emb3.reshape(_NC * N_NODES, _COLS))

    light = jnp.concatenate([m2[:N_NODES], m2[N_NODES:]], axis=1)
    return (light[:NUM_USER], light[NUM_USER:])


# speedup vs baseline: 7.1273x; 1.3663x over previous
import functools

import jax
import jax.numpy as jnp
from jax import lax
from jax.experimental import pallas as pl
from jax.experimental.pallas import tpu as pltpu
from jax.experimental.pallas import tpu_sc as plsc

NUM_USER = 10000
NUM_ITEM = 40000
N_NODES = NUM_USER + NUM_ITEM
DIM_FEAT = 512
DIM_LATENT = 64
N_EDGES = 800000
N_LAYERS = 3

_NC = 2
_NS = 16
_COLS = DIM_LATENT // _NC
_SLAB = 3136
_ACC_ROWS = _NS * _SLAB
_W = 128
_N_SUP = N_EDGES // (2 * _W)


def _propagate(emb, src3, dst3, wgt3):
    mesh = plsc.VectorSubcoreMesh(
        core_axis_name="c", subcore_axis_name="s",
        num_cores=_NC, num_subcores=_NS)

    @functools.partial(
        pl.kernel,
        out_type=jax.ShapeDtypeStruct((_NC, N_NODES, _COLS), jnp.float32),
        mesh=mesh,
        compiler_params=pltpu.CompilerParams(use_tc_tiling_on_sc=False),
        scratch_types=[
            pltpu.VMEM_SHARED((_ACC_ROWS, _COLS), jnp.float32),
            pltpu.VMEM((2, 2, _W), jnp.int32),
            pltpu.VMEM((2, 2, _W), jnp.int32),
            pltpu.VMEM((2, 2, _W), jnp.float32),
            pltpu.VMEM((2, _W, _COLS), jnp.float32),
            pltpu.VMEM((_W, _COLS), jnp.float32),
            pltpu.SemaphoreType.DMA,
            pltpu.SemaphoreType.DMA,
            pltpu.SemaphoreType.DMA,
            pltpu.SemaphoreType.DMA,
            pltpu.SemaphoreType.DMA,
        ],
    )
    def k(emb_h, src_h, dst_h, wgt_h, out_h, acc, srcb, dstb, wgtb, rows,
          zbuf, semg0, semg1, sems0, sems1, semi):
        semg = (semg0, semg1)
        c = lax.axis_index("c")
        s = lax.axis_index("s")
        my_emb = emb_h.at[c]
        my_out = out_h.at[c]
        base = s * _SLAB

        def _zrow(r, carry):
            for q in range(_COLS // 16):
                zbuf[r, pl.ds(q * 16, 16)] = jnp.zeros((16,), jnp.float32)
            return carry
        lax.fori_loop(0, _W, _zrow, 0)

        def _zcp(i, carry):
            pltpu.sync_copy(zbuf, acc.at[pl.ds(base + i * _W, _W)])
            return carry
        lax.fori_loop(0, _SLAB // _W, _zcp, 0)
        pltpu.sync_copy(zbuf.at[pl.ds(0, _SLAB % _W)],
                        acc.at[pl.ds(base + (_SLAB // _W) * _W, _SLAB % _W)])
        plsc.subcore_barrier()

        n_sup = (_N_SUP - s + _NS - 1) // _NS

        def _idx_load(kk2, b2):
            sup = s + kk2 * _NS
            pltpu.sync_copy(src_h.at[sup], srcb.at[b2])
            pltpu.sync_copy(dst_h.at[sup], dstb.at[b2])
            pltpu.sync_copy(wgt_h.at[sup], wgtb.at[b2])

        def _g_issue(b2, j):
            pltpu.async_copy(my_emb.at[srcb.at[b2, j]], rows.at[j], semg[j])

        def _g_wait(b2, j):
            pltpu.make_async_copy(
                my_emb.at[srcb.at[b2, j]], rows.at[j], semg[j]).wait()

        def _compute(b2, j):
            def _wmul(g, carry2):
                w16 = wgtb[b2, j, pl.ds(g * 16, 16)]
                for eo in range(16):
                    e = g * 16 + eo
                    wv = w16[eo]
                    for q in range(_COLS // 16):
                        rows[j, e, pl.ds(q * 16, 16)] = (
                            rows[j, e, pl.ds(q * 16, 16)] * wv)
                return carry2
            lax.fori_loop(0, _W // 16, _wmul, 0)

        _idx_load(0, 0)
        _g_issue(0, 0)

        def _sup2(kk4, carry):
            for b2 in range(2):
                kk2 = kk4 * 2 + b2

                @pl.when(kk2 < n_sup)
                def _():
                    sup_n = jnp.minimum(s + (kk2 + 1) * _NS, _N_SUP - 1)
                    di0 = pltpu.async_copy(
                        src_h.at[sup_n], srcb.at[1 - b2], semi)
                    di1 = pltpu.async_copy(
                        dst_h.at[sup_n], dstb.at[1 - b2], semi)
                    di2 = pltpu.async_copy(
                        wgt_h.at[sup_n], wgtb.at[1 - b2], semi)
                    _g_wait(b2, 0)
                    _g_issue(b2, 1)
                    _compute(b2, 0)
                    sd0 = pltpu.async_copy(
                        rows.at[0], acc.at[dstb.at[b2, 0]], sems0, add=True)
                    _g_wait(b2, 1)
                    _compute(b2, 1)
                    sd0.wait()
                    di0.wait()
                    di1.wait()
                    di2.wait()

                    @pl.when(kk2 + 1 < n_sup)
                    def _():
                        _g_issue(1 - b2, 0)
                    sd1 = pltpu.async_copy(
                        rows.at[1], acc.at[dstb.at[b2, 1]], sems1, add=True)
                    sd1.wait()
            return carry
        lax.fori_loop(0, (n_sup + 1) // 2, _sup2, 0)
        plsc.subcore_barrier()

        n_full = jnp.where(s < _NS - 1, _SLAB // _W, 23)

        def _wb(i, carry):
            pltpu.sync_copy(acc.at[pl.ds(base + i * _W, _W)],
                            my_out.at[pl.ds(base + i * _W, _W)])
            return carry
        lax.fori_loop(0, n_full, _wb, 0)

        @pl.when(s < _NS - 1)
        def _tail_a():
            pltpu.sync_copy(acc.at[pl.ds(base + 24 * _W, 64)],
                            my_out.at[pl.ds(base + 24 * _W, 64)])

        @pl.when(s == _NS - 1)
        def _tail_b():
            pltpu.sync_copy(acc.at[pl.ds(base + 23 * _W, 16)],
                            my_out.at[pl.ds(base + 23 * _W, 16)])

    return k(emb, src3, dst3, wgt3)


_BM = 2000


def _item_body(f_ref, wt_ref, b_ref, o_ref):
    x = jnp.dot(f_ref[...], wt_ref[...], preferred_element_type=jnp.float32)
    x = x + b_ref[...]
    norm = jnp.sqrt(jnp.sum(x * x, axis=1, keepdims=True))
    o_ref[...] = x / jnp.maximum(norm, 1e-12)


def _user_body(u_ref, o_ref):
    x = u_ref[...]
    norm = jnp.sqrt(jnp.sum(x * x, axis=1, keepdims=True))
    o_ref[...] = x / jnp.maximum(norm, 1e-12)


def _mean_body(a_ref, b_ref, c_ref, d_ref, o_ref):
    o_ref[...] = 0.25 * (a_ref[...] + b_ref[...] + c_ref[...] + d_ref[...])


def kernel(features, user_id_preference, edge_index, edge_weight, W, b):
    Wt = W.T
    b2 = b.reshape(1, DIM_LATENT)

    items0 = pl.pallas_call(
        _item_body,
        grid=(NUM_ITEM // _BM,),
        in_specs=[
            pl.BlockSpec((_BM, DIM_FEAT), lambda i: (i, 0)),
            pl.BlockSpec((DIM_FEAT, DIM_LATENT), lambda i: (0, 0)),
            pl.BlockSpec((1, DIM_LATENT), lambda i: (0, 0)),
        ],
        out_specs=pl.BlockSpec((_BM, DIM_LATENT), lambda i: (i, 0)),
        out_shape=jax.ShapeDtypeStruct((NUM_ITEM, DIM_LATENT), jnp.float32),
    )(features, Wt, b2)

    users0 = pl.pallas_call(
        _user_body,
        out_shape=jax.ShapeDtypeStruct((NUM_USER, DIM_LATENT), jnp.float32),
    )(user_id_preference)

    emb0 = jnp.concatenate([users0, items0], axis=0)
    emb0_ab = jnp.stack([emb0[:, :_COLS], emb0[:, _COLS:]], axis=0)

    src3 = edge_index[0].reshape(_N_SUP, 2, _W)
    dst3 = edge_index[1].reshape(_N_SUP, 2, _W)
    wgt3 = edge_weight.reshape(_N_SUP, 2, _W)
    emb1 = _propagate(emb0_ab, src3, dst3, wgt3)
    emb2 = _propagate(emb1, src3, dst3, wgt3)
    emb3 = _propagate(emb2, src3, dst3, wgt3)

    _BR = 2000
    m2 = pl.pallas_call(
        _mean_body,
        grid=(_NC * N_NODES // _BR,),
        in_specs=[pl.BlockSpec((_BR, _COLS), lambda i: (i, 0))] * 4,
        out_specs=pl.BlockSpec((_BR, _COLS), lambda i: (i, 0)),
        out_shape=jax.ShapeDtypeStruct((_NC * N_NODES, _COLS), jnp.float32),
    )(emb0_ab.reshape(_NC * N_NODES, _COLS),
      emb1.reshape(_NC * N_NODES, _COLS),
      emb2.reshape(_NC * N_NODES, _COLS),
      emb3.reshape(_NC * N_NODES, _COLS))

    light = jnp.concatenate([m2[:N_NODES], m2[N_NODES:]], axis=1)
    return (light[:NUM_USER], light[NUM_USER:])


# speedup vs baseline: 8.1271x; 1.1403x over previous
import functools

import jax
import jax.numpy as jnp
from jax import lax
from jax.experimental import pallas as pl
from jax.experimental.pallas import tpu as pltpu
from jax.experimental.pallas import tpu_sc as plsc

NUM_USER = 10000
NUM_ITEM = 40000
N_NODES = NUM_USER + NUM_ITEM
DIM_FEAT = 512
DIM_LATENT = 64
N_EDGES = 800000
N_LAYERS = 3

_NC = 2
_NS = 16
_COLS = DIM_LATENT // _NC
_SLAB = 3136
_ACC_ROWS = _NS * _SLAB
_W = 128
_N_SUP = N_EDGES // (2 * _W)


def _propagate(emb, src3, dst3, wgt3):
    mesh = plsc.VectorSubcoreMesh(
        core_axis_name="c", subcore_axis_name="s",
        num_cores=_NC, num_subcores=_NS)

    @functools.partial(
        pl.kernel,
        out_type=jax.ShapeDtypeStruct((_NC, N_NODES, _COLS), jnp.float32),
        mesh=mesh,
        compiler_params=pltpu.CompilerParams(use_tc_tiling_on_sc=False),
        scratch_types=[
            pltpu.VMEM_SHARED((_ACC_ROWS, _COLS), jnp.float32),
            pltpu.VMEM((2, 2, _W), jnp.int32),
            pltpu.VMEM((2, 2, _W), jnp.int32),
            pltpu.VMEM((2, 2, _W), jnp.float32),
            pltpu.VMEM((2, 2, _W, _COLS), jnp.float32),
            pltpu.VMEM((_W, _COLS), jnp.float32),
            pltpu.SemaphoreType.DMA,
            pltpu.SemaphoreType.DMA,
            pltpu.SemaphoreType.DMA,
            pltpu.SemaphoreType.DMA,
            pltpu.SemaphoreType.DMA,
        ],
    )
    def k(emb_h, src_h, dst_h, wgt_h, out_h, acc, srcb, dstb, wgtb, rows,
          zbuf, semg0, semg1, sems0, sems1, semi):
        semg = (semg0, semg1)
        c = lax.axis_index("c")
        s = lax.axis_index("s")
        my_emb = emb_h.at[c]
        my_out = out_h.at[c]
        base = s * _SLAB

        def _zrow(r, carry):
            for q in range(_COLS // 16):
                zbuf[r, pl.ds(q * 16, 16)] = jnp.zeros((16,), jnp.float32)
            return carry
        lax.fori_loop(0, _W, _zrow, 0)

        def _zcp(i, carry):
            pltpu.sync_copy(zbuf, acc.at[pl.ds(base + i * _W, _W)])
            return carry
        lax.fori_loop(0, _SLAB // _W, _zcp, 0)
        pltpu.sync_copy(zbuf.at[pl.ds(0, _SLAB % _W)],
                        acc.at[pl.ds(base + (_SLAB // _W) * _W, _SLAB % _W)])
        plsc.subcore_barrier()

        n_sup = (_N_SUP - s + _NS - 1) // _NS

        def _idx_load(kk2, b2):
            sup = s + kk2 * _NS
            pltpu.sync_copy(src_h.at[sup], srcb.at[b2])
            pltpu.sync_copy(dst_h.at[sup], dstb.at[b2])
            pltpu.sync_copy(wgt_h.at[sup], wgtb.at[b2])

        def _g_issue(b2, j):
            pltpu.async_copy(
                my_emb.at[srcb.at[b2, j]], rows.at[b2, j], semg[j])

        def _g_wait(b2, j):
            pltpu.make_async_copy(
                my_emb.at[srcb.at[b2, j]], rows.at[b2, j], semg[j]).wait()

        def _compute(b2, j):
            def _wmul(g, carry2):
                w16 = wgtb[b2, j, pl.ds(g * 16, 16)]
                for eo in range(16):
                    e = g * 16 + eo
                    wv = w16[eo]
                    for q in range(_COLS // 16):
                        rows[b2, j, e, pl.ds(q * 16, 16)] = (
                            rows[b2, j, e, pl.ds(q * 16, 16)] * wv)
                return carry2
            lax.fori_loop(0, _W // 16, _wmul, 0)

        _idx_load(0, 0)
        _g_issue(0, 0)
        _g_issue(0, 1)

        def _sup2(kk4, carry):
            for b2 in range(2):
                kk2 = kk4 * 2 + b2

                @pl.when(kk2 < n_sup)
                def _():
                    sup_n = jnp.minimum(s + (kk2 + 1) * _NS, _N_SUP - 1)
                    di0 = pltpu.async_copy(
                        src_h.at[sup_n], srcb.at[1 - b2], semi)
                    di1 = pltpu.async_copy(
                        dst_h.at[sup_n], dstb.at[1 - b2], semi)
                    di2 = pltpu.async_copy(
                        wgt_h.at[sup_n], wgtb.at[1 - b2], semi)
                    _g_wait(b2, 0)
                    _compute(b2, 0)
                    sd0 = pltpu.async_copy(
                        rows.at[b2, 0], acc.at[dstb.at[b2, 0]], sems0,
                        add=True)
                    _g_wait(b2, 1)
                    _compute(b2, 1)
                    sd1 = pltpu.async_copy(
                        rows.at[b2, 1], acc.at[dstb.at[b2, 1]], sems1,
                        add=True)
                    di0.wait()
                    di1.wait()
                    di2.wait()
                    sd0.wait()
                    sd1.wait()

                    @pl.when(kk2 + 1 < n_sup)
                    def _():
                        _g_issue(1 - b2, 0)
                        _g_issue(1 - b2, 1)
            return carry
        lax.fori_loop(0, (n_sup + 1) // 2, _sup2, 0)
        plsc.subcore_barrier()

        n_full = jnp.where(s < _NS - 1, _SLAB // _W, 23)

        def _wb(i, carry):
            pltpu.sync_copy(acc.at[pl.ds(base + i * _W, _W)],
                            my_out.at[pl.ds(base + i * _W, _W)])
            return carry
        lax.fori_loop(0, n_full, _wb, 0)

        @pl.when(s < _NS - 1)
        def _tail_a():
            pltpu.sync_copy(acc.at[pl.ds(base + 24 * _W, 64)],
                            my_out.at[pl.ds(base + 24 * _W, 64)])

        @pl.when(s == _NS - 1)
        def _tail_b():
            pltpu.sync_copy(acc.at[pl.ds(base + 23 * _W, 16)],
                            my_out.at[pl.ds(base + 23 * _W, 16)])

    return k(emb, src3, dst3, wgt3)


_BM = 2000


def _item_body(f_ref, wt_ref, b_ref, o_ref):
    x = jnp.dot(f_ref[...], wt_ref[...], preferred_element_type=jnp.float32)
    x = x + b_ref[...]
    norm = jnp.sqrt(jnp.sum(x * x, axis=1, keepdims=True))
    o_ref[...] = x / jnp.maximum(norm, 1e-12)


def _user_body(u_ref, o_ref):
    x = u_ref[...]
    norm = jnp.sqrt(jnp.sum(x * x, axis=1, keepdims=True))
    o_ref[...] = x / jnp.maximum(norm, 1e-12)


def _mean_body(a_ref, b_ref, c_ref, d_ref, o_ref):
    o_ref[...] = 0.25 * (a_ref[...] + b_ref[...] + c_ref[...] + d_ref[...])


def kernel(features, user_id_preference, edge_index, edge_weight, W, b):
    Wt = W.T
    b2 = b.reshape(1, DIM_LATENT)

    items0 = pl.pallas_call(
        _item_body,
        grid=(NUM_ITEM // _BM,),
        in_specs=[
            pl.BlockSpec((_BM, DIM_FEAT), lambda i: (i, 0)),
            pl.BlockSpec((DIM_FEAT, DIM_LATENT), lambda i: (0, 0)),
            pl.BlockSpec((1, DIM_LATENT), lambda i: (0, 0)),
        ],
        out_specs=pl.BlockSpec((_BM, DIM_LATENT), lambda i: (i, 0)),
        out_shape=jax.ShapeDtypeStruct((NUM_ITEM, DIM_LATENT), jnp.float32),
    )(features, Wt, b2)

    users0 = pl.pallas_call(
        _user_body,
        out_shape=jax.ShapeDtypeStruct((NUM_USER, DIM_LATENT), jnp.float32),
    )(user_id_preference)

    emb0 = jnp.concatenate([users0, items0], axis=0)
    emb0_ab = jnp.stack([emb0[:, :_COLS], emb0[:, _COLS:]], axis=0)

    src3 = edge_index[0].reshape(_N_SUP, 2, _W)
    dst3 = edge_index[1].reshape(_N_SUP, 2, _W)
    wgt3 = edge_weight.reshape(_N_SUP, 2, _W)
    emb1 = _propagate(emb0_ab, src3, dst3, wgt3)
    emb2 = _propagate(emb1, src3, dst3, wgt3)
    emb3 = _propagate(emb2, src3, dst3, wgt3)

    _BR = 2000
    m2 = pl.pallas_call(
        _mean_body,
        grid=(_NC * N_NODES // _BR,),
        in_specs=[pl.BlockSpec((_BR, _COLS), lambda i: (i, 0))] * 4,
        out_specs=pl.BlockSpec((_BR, _COLS), lambda i: (i, 0)),
        out_shape=jax.ShapeDtypeStruct((_NC * N_NODES, _COLS), jnp.float32),
    )(emb0_ab.reshape(_NC * N_NODES, _COLS),
      emb1.reshape(_NC * N_NODES, _COLS),
      emb2.reshape(_NC * N_NODES, _COLS),
      emb3.reshape(_NC * N_NODES, _COLS))

    light = jnp.concatenate([m2[:N_NODES], m2[N_NODES:]], axis=1)
    return (light[:NUM_USER], light[NUM_USER:])


# speedup vs baseline: 8.4446x; 1.0391x over previous
import functools

import jax
import jax.numpy as jnp
from jax import lax
from jax.experimental import pallas as pl
from jax.experimental.pallas import tpu as pltpu
from jax.experimental.pallas import tpu_sc as plsc

NUM_USER = 10000
NUM_ITEM = 40000
N_NODES = NUM_USER + NUM_ITEM
DIM_FEAT = 512
DIM_LATENT = 64
N_EDGES = 800000
N_LAYERS = 3

_NC = 2
_NS = 16
_COLS = DIM_LATENT // _NC
_SLAB = 3136
_ACC_ROWS = _NS * _SLAB
_W = 128
_N_SUP = N_EDGES // (2 * _W)


def _propagate(emb, src3, dst3, wgt3):
    mesh = plsc.VectorSubcoreMesh(
        core_axis_name="c", subcore_axis_name="s",
        num_cores=_NC, num_subcores=_NS)

    @functools.partial(
        pl.kernel,
        out_type=jax.ShapeDtypeStruct((_NC, N_NODES, _COLS), jnp.float32),
        mesh=mesh,
        compiler_params=pltpu.CompilerParams(use_tc_tiling_on_sc=False),
        scratch_types=[
            pltpu.VMEM_SHARED((_ACC_ROWS, _COLS), jnp.float32),
            pltpu.VMEM((2, 2, _W), jnp.int32),
            pltpu.VMEM((2, 2, _W), jnp.int32),
            pltpu.VMEM((2, 2, _W), jnp.float32),
            pltpu.VMEM((2, 2, _W, _COLS), jnp.float32),
            pltpu.VMEM((_W, _COLS), jnp.float32),
            pltpu.SemaphoreType.DMA,
            pltpu.SemaphoreType.DMA,
            pltpu.SemaphoreType.DMA,
            pltpu.SemaphoreType.DMA,
            pltpu.SemaphoreType.DMA,
        ],
    )
    def k(emb_h, src_h, dst_h, wgt_h, out_h, acc, srcb, dstb, wgtb, rows,
          zbuf, semg0, semg1, sems0, sems1, semi):
        semg = (semg0, semg1)
        c = lax.axis_index("c")
        s = lax.axis_index("s")
        my_emb = emb_h.at[c]
        my_out = out_h.at[c]
        base = s * _SLAB

        def _zrow(r, carry):
            for q in range(_COLS // 16):
                zbuf[r, pl.ds(q * 16, 16)] = jnp.zeros((16,), jnp.float32)
            return carry
        lax.fori_loop(0, _W, _zrow, 0)

        def _zcp(i, carry):
            pltpu.sync_copy(zbuf, acc.at[pl.ds(base + i * _W, _W)])
            return carry
        lax.fori_loop(0, _SLAB // _W, _zcp, 0)
        pltpu.sync_copy(zbuf.at[pl.ds(0, _SLAB % _W)],
                        acc.at[pl.ds(base + (_SLAB // _W) * _W, _SLAB % _W)])
        plsc.subcore_barrier()

        n_sup = (_N_SUP - s + _NS - 1) // _NS

        def _idx_load(kk2, b2):
            sup = s + kk2 * _NS
            pltpu.sync_copy(src_h.at[sup], srcb.at[b2])
            pltpu.sync_copy(dst_h.at[sup], dstb.at[b2])
            pltpu.sync_copy(wgt_h.at[sup], wgtb.at[b2])

        def _g_issue(b2, j):
            pltpu.async_copy(
                my_emb.at[srcb.at[b2, j]], rows.at[b2, j], semg[j])

        def _g_wait(b2, j):
            pltpu.make_async_copy(
                my_emb.at[srcb.at[b2, j]], rows.at[b2, j], semg[j]).wait()

        def _compute(b2, j):
            def _wmul(g, carry2):
                w16 = wgtb[b2, j, pl.ds(g * 16, 16)]
                for eo in range(16):
                    e = g * 16 + eo
                    wv = w16[eo]
                    for q in range(_COLS // 16):
                        rows[b2, j, e, pl.ds(q * 16, 16)] = (
                            rows[b2, j, e, pl.ds(q * 16, 16)] * wv)
                return carry2
            lax.fori_loop(0, _W // 16, _wmul, 0)

        _idx_load(0, 0)
        _g_issue(0, 0)
        _g_issue(0, 1)

        def _sup2(kk4, carry):
            for b2 in range(2):
                kk2 = kk4 * 2 + b2

                @pl.when(kk2 < n_sup)
                def _():
                    sup_n = jnp.minimum(s + (kk2 + 1) * _NS, _N_SUP - 1)
                    di0 = pltpu.async_copy(
                        src_h.at[sup_n], srcb.at[1 - b2], semi)
                    di1 = pltpu.async_copy(
                        dst_h.at[sup_n], dstb.at[1 - b2], semi)
                    di2 = pltpu.async_copy(
                        wgt_h.at[sup_n], wgtb.at[1 - b2], semi)
                    _g_wait(b2, 0)
                    _compute(b2, 0)
                    sd0 = pltpu.async_copy(
                        rows.at[b2, 0], acc.at[dstb.at[b2, 0]], sems0,
                        add=True)
                    _g_wait(b2, 1)
                    _compute(b2, 1)
                    sd1 = pltpu.async_copy(
                        rows.at[b2, 1], acc.at[dstb.at[b2, 1]], sems1,
                        add=True)
                    di0.wait()
                    di1.wait()
                    di2.wait()
                    sd0.wait()
                    sd1.wait()

                    @pl.when(kk2 + 1 < n_sup)
                    def _():
                        _g_issue(1 - b2, 0)
                        _g_issue(1 - b2, 1)
            return carry
        lax.fori_loop(0, (n_sup + 1) // 2, _sup2, 0)
        plsc.subcore_barrier()

        n_full = jnp.where(s < _NS - 1, _SLAB // _W, 23)

        def _wb(i, carry):
            pltpu.sync_copy(acc.at[pl.ds(base + i * _W, _W)],
                            my_out.at[pl.ds(base + i * _W, _W)])
            return carry
        lax.fori_loop(0, n_full, _wb, 0)

        @pl.when(s < _NS - 1)
        def _tail_a():
            pltpu.sync_copy(acc.at[pl.ds(base + 24 * _W, 64)],
                            my_out.at[pl.ds(base + 24 * _W, 64)])

        @pl.when(s == _NS - 1)
        def _tail_b():
            pltpu.sync_copy(acc.at[pl.ds(base + 23 * _W, 16)],
                            my_out.at[pl.ds(base + 23 * _W, 16)])

    return k(emb, src3, dst3, wgt3)


_BM = 2000


def _item_body(f_ref, wt_ref, b_ref, o_ref):
    x = jnp.dot(f_ref[...], wt_ref[...], preferred_element_type=jnp.float32)
    x = x + b_ref[...]
    norm = jnp.sqrt(jnp.sum(x * x, axis=1, keepdims=True))
    o_ref[...] = x / jnp.maximum(norm, 1e-12)


def _user_body(u_ref, o_ref):
    x = u_ref[...]
    norm = jnp.sqrt(jnp.sum(x * x, axis=1, keepdims=True))
    o_ref[...] = x / jnp.maximum(norm, 1e-12)


def _mean_body(a_ref, b_ref, c_ref, d_ref, o_ref):
    o_ref[...] = 0.25 * (a_ref[...] + b_ref[...] + c_ref[...] + d_ref[...])


def kernel(features, user_id_preference, edge_index, edge_weight, W, b):
    Wt = W.T
    b2 = b.reshape(1, DIM_LATENT)

    items0 = pl.pallas_call(
        _item_body,
        grid=(NUM_ITEM // _BM,),
        in_specs=[
            pl.BlockSpec((_BM, DIM_FEAT), lambda i: (i, 0)),
            pl.BlockSpec((DIM_FEAT, DIM_LATENT), lambda i: (0, 0)),
            pl.BlockSpec((1, DIM_LATENT), lambda i: (0, 0)),
        ],
        out_specs=pl.BlockSpec((_BM, DIM_LATENT), lambda i: (i, 0)),
        out_shape=jax.ShapeDtypeStruct((NUM_ITEM, DIM_LATENT), jnp.float32),
    )(features, Wt, b2)

    users0 = pl.pallas_call(
        _user_body,
        out_shape=jax.ShapeDtypeStruct((NUM_USER, DIM_LATENT), jnp.float32),
    )(user_id_preference)

    emb0 = jnp.concatenate([users0, items0], axis=0)
    emb0_ab = jnp.stack([emb0[:, :_COLS], emb0[:, _COLS:]], axis=0)

    src3 = edge_index[0].reshape(_N_SUP, 2, _W)
    dst3 = edge_index[1].reshape(_N_SUP, 2, _W)
    wgt3 = edge_weight.reshape(_N_SUP, 2, _W)
    emb1 = _propagate(emb0_ab, src3, dst3, wgt3)
    emb2 = _propagate(emb1, src3, dst3, wgt3)
    emb3 = _propagate(emb2, src3, dst3, wgt3)

    _MR = _NC * N_NODES * _COLS // 128
    _BR = 1000
    m2 = pl.pallas_call(
        _mean_body,
        grid=(_MR // _BR,),
        in_specs=[pl.BlockSpec((_BR, 128), lambda i: (i, 0))] * 4,
        out_specs=pl.BlockSpec((_BR, 128), lambda i: (i, 0)),
        out_shape=jax.ShapeDtypeStruct((_MR, 128), jnp.float32),
    )(emb0_ab.reshape(_MR, 128),
      emb1.reshape(_MR, 128),
      emb2.reshape(_MR, 128),
      emb3.reshape(_MR, 128))

    m2 = m2.reshape(_NC * N_NODES, _COLS)
    light = jnp.concatenate([m2[:N_NODES], m2[N_NODES:]], axis=1)
    return (light[:NUM_USER], light[NUM_USER:])
